# Initial kernel scaffold; baseline (speedup 1.0000x reference)
import jax
import jax.numpy as jnp
from jax.experimental import pallas as pl


def kernel(idx, noise, centers_raw, log_scales, theta_raw, logits):
    raise NotImplementedError("write your pallas kernel here")


# speedup vs baseline: 359.3789x; 359.3789x over previous
import functools
import math

import jax
import jax.numpy as jnp
from jax import lax
from jax.experimental import pallas as pl
from jax.experimental.pallas import tpu as pltpu
from jax.experimental.pallas import tpu_sc as plsc

K = 64
B = 64
N = 32768
MIN_SCALE = 0.0005
LOG_2PI = math.log(2.0 * math.pi)

NC = 2
NS = 16
L = 16
NW = NC * NS
ROWS_PER_W = B // NW
CHUNK = 8192
CPR = N // CHUNK
NCH = ROWS_PER_W * CPR

CBLK = 8192


def _tc_body(ls0_ref, ls1_ref, logits_ref, noise_ref, c_ref, part_ref):
    i = pl.program_id(0)

    @pl.when(i == 0)
    def _init():
        ls0 = ls0_ref[...]
        ls1 = ls1_ref[...]
        lg = logits_ref[...]
        sp0 = jnp.maximum(ls0, 0.0) + jnp.log(1.0 + jnp.exp(-jnp.abs(ls0))) + MIN_SCALE
        sp1 = jnp.maximum(ls1, 0.0) + jnp.log(1.0 + jnp.exp(-jnp.abs(ls1))) + MIN_SCALE
        logdet = jnp.log(sp0 * sp1 + 1e-12)
        e = jnp.exp(lg - jnp.max(lg))
        w = e / jnp.sum(e)
        logw = jnp.log(w + 1e-9)
        c_ref[...] = logw - logdet
        part_ref[...] = jnp.full((B,), -N * LOG_2PI, jnp.float32)

    x = noise_ref[...]
    part_ref[...] = part_ref[...] - 0.5 * jnp.sum(x * x, axis=1)


_tc_call = pl.pallas_call(
    _tc_body,
    grid=(2 * N // CBLK,),
    in_specs=[
        pl.BlockSpec((K,), lambda i: (0,)),
        pl.BlockSpec((K,), lambda i: (0,)),
        pl.BlockSpec((K,), lambda i: (0,)),
        pl.BlockSpec((B, CBLK), lambda i: (0, i)),
    ],
    out_specs=[
        pl.BlockSpec((K,), lambda i: (0,)),
        pl.BlockSpec((B,), lambda i: (0,)),
    ],
    out_shape=[
        jax.ShapeDtypeStruct((K,), jnp.float32),
        jax.ShapeDtypeStruct((B,), jnp.float32),
    ],
)

_sc_mesh = plsc.VectorSubcoreMesh(core_axis_name="c", subcore_axis_name="s")


@functools.partial(
    pl.kernel,
    out_type=jax.ShapeDtypeStruct((B, L), jnp.float32),
    mesh=_sc_mesh,
    compiler_params=pltpu.CompilerParams(needs_layout_passes=False),
    scratch_types=[
        pltpu.VMEM((CHUNK,), jnp.int32),
        pltpu.VMEM((CHUNK,), jnp.int32),
        pltpu.VMEM((K,), jnp.float32),
        pltpu.VMEM((B,), jnp.float32),
        pltpu.VMEM((ROWS_PER_W, L), jnp.float32),
        pltpu.SemaphoreType.DMA,
        pltpu.SemaphoreType.DMA,
    ],
)
def _sc_gather(c_hbm, part_hbm, idx_hbm, out_hbm,
               buf0, buf1, c_v, part_v, res_v, sem0, sem1):
    wid = lax.axis_index("s") * NC + lax.axis_index("c")
    r0 = wid * ROWS_PER_W

    pltpu.sync_copy(c_hbm, c_v)
    pltpu.sync_copy(part_hbm, part_v)

    bufs = (buf0, buf1)
    sems = (sem0, sem1)

    def start(t):
        r = r0 + (t // CPR)
        off = (t % CPR) * CHUNK
        return pltpu.async_copy(idx_hbm.at[r, pl.ds(off, CHUNK)],
                                bufs[t % 2], sems[t % 2])

    handles = {0: start(0)}
    for rloc in range(ROWS_PER_W):
        zero = jnp.zeros((L,), jnp.float32)
        accs = (zero, zero, zero, zero)
        for kk in range(CPR):
            t = rloc * CPR + kk
            if t + 1 < NCH:
                handles[t + 1] = start(t + 1)
            handles[t].wait()
            buf = bufs[t % 2]

            @plsc.parallel_loop(0, CHUNK, step=4 * L, unroll=2, carry=accs)
            def _acc(p, a):
                g0 = plsc.load_gather(c_v, [buf[pl.ds(p, L)]])
                g1 = plsc.load_gather(c_v, [buf[pl.ds(p + L, L)]])
                g2 = plsc.load_gather(c_v, [buf[pl.ds(p + 2 * L, L)]])
                g3 = plsc.load_gather(c_v, [buf[pl.ds(p + 3 * L, L)]])
                return (a[0] + g0, a[1] + g1, a[2] + g2, a[3] + g3)

            accs = _acc
        total = accs[0] + accs[1] + accs[2] + accs[3]
        part_vec = plsc.load_gather(
            part_v, [jnp.full((L,), r0 + rloc, jnp.int32)])
        res_v[rloc, :] = jnp.full((L,), jnp.sum(total), jnp.float32) + part_vec
    pltpu.sync_copy(res_v, out_hbm.at[pl.ds(r0, ROWS_PER_W)])


def kernel(idx, noise, centers_raw, log_scales, theta_raw, logits):
    del centers_raw, theta_raw
    noise2 = noise.reshape(B, 2 * N)
    c, part = _tc_call(log_scales[:, 0], log_scales[:, 1], logits, noise2)
    out2d = _sc_gather(c, part, idx)
    return out2d[:, 0]


# speedup vs baseline: 749.1169x; 2.0845x over previous
import functools
import math

import jax
import jax.numpy as jnp
from jax import lax
from jax.experimental import pallas as pl
from jax.experimental.pallas import tpu as pltpu
from jax.experimental.pallas import tpu_sc as plsc

K = 64
B = 64
N = 32768
MIN_SCALE = 0.0005
LOG_2PI = math.log(2.0 * math.pi)

NC = 2
NS = 16
L = 16
NW = NC * NS
ROWS_PER_W = B // NW
CHUNK = 8192
CPR = N // CHUNK
NCH = ROWS_PER_W * CPR

CBLK = 8192


def _tc_body(ls0_ref, ls1_ref, logits_ref, noise_ref, c_ref, part_ref):
    i = pl.program_id(0)

    @pl.when(i == 0)
    def _init():
        ls0 = ls0_ref[...]
        ls1 = ls1_ref[...]
        lg = logits_ref[...]
        sp0 = jnp.maximum(ls0, 0.0) + jnp.log(1.0 + jnp.exp(-jnp.abs(ls0))) + MIN_SCALE
        sp1 = jnp.maximum(ls1, 0.0) + jnp.log(1.0 + jnp.exp(-jnp.abs(ls1))) + MIN_SCALE
        logdet = jnp.log(sp0 * sp1 + 1e-12)
        e = jnp.exp(lg - jnp.max(lg))
        w = e / jnp.sum(e)
        logw = jnp.log(w + 1e-9)
        c_ref[...] = logw - logdet
        part_ref[...] = jnp.full((B,), -N * LOG_2PI, jnp.float32)

    x = noise_ref[...]
    part_ref[...] = part_ref[...] - 0.5 * jnp.sum(x * x, axis=(1, 2))


_RB = CBLK // 128

_tc_call = pl.pallas_call(
    _tc_body,
    grid=(2 * N // CBLK,),
    in_specs=[
        pl.BlockSpec((K,), lambda i: (0,)),
        pl.BlockSpec((K,), lambda i: (0,)),
        pl.BlockSpec((K,), lambda i: (0,)),
        pl.BlockSpec((B, _RB, 128), lambda i: (0, i, 0)),
    ],
    out_specs=[
        pl.BlockSpec((K,), lambda i: (0,)),
        pl.BlockSpec((B,), lambda i: (0,)),
    ],
    out_shape=[
        jax.ShapeDtypeStruct((K,), jnp.float32),
        jax.ShapeDtypeStruct((B,), jnp.float32),
    ],
)

_sc_mesh = plsc.VectorSubcoreMesh(core_axis_name="c", subcore_axis_name="s")


@functools.partial(
    pl.kernel,
    out_type=jax.ShapeDtypeStruct((B, L), jnp.float32),
    mesh=_sc_mesh,
    compiler_params=pltpu.CompilerParams(needs_layout_passes=False),
    scratch_types=[
        pltpu.VMEM((CHUNK,), jnp.int32),
        pltpu.VMEM((CHUNK,), jnp.int32),
        pltpu.VMEM((K,), jnp.float32),
        pltpu.VMEM((B,), jnp.float32),
        pltpu.VMEM((ROWS_PER_W, L), jnp.float32),
        pltpu.SemaphoreType.DMA,
        pltpu.SemaphoreType.DMA,
    ],
)
def _sc_gather(c_hbm, part_hbm, idx_hbm, out_hbm,
               buf0, buf1, c_v, part_v, res_v, sem0, sem1):
    wid = lax.axis_index("s") * NC + lax.axis_index("c")
    r0 = wid * ROWS_PER_W

    pltpu.sync_copy(c_hbm, c_v)
    pltpu.sync_copy(part_hbm, part_v)

    bufs = (buf0, buf1)
    sems = (sem0, sem1)

    def start(t):
        r = r0 + (t // CPR)
        off = (t % CPR) * CHUNK
        return pltpu.async_copy(idx_hbm.at[r, pl.ds(off, CHUNK)],
                                bufs[t % 2], sems[t % 2])

    handles = {0: start(0)}
    for rloc in range(ROWS_PER_W):
        zero = jnp.zeros((L,), jnp.float32)
        accs = (zero, zero, zero, zero)
        for kk in range(CPR):
            t = rloc * CPR + kk
            if t + 1 < NCH:
                handles[t + 1] = start(t + 1)
            handles[t].wait()
            buf = bufs[t % 2]

            @plsc.parallel_loop(0, CHUNK, step=4 * L, unroll=2, carry=accs)
            def _acc(p, a):
                g0 = plsc.load_gather(c_v, [buf[pl.ds(p, L)]])
                g1 = plsc.load_gather(c_v, [buf[pl.ds(p + L, L)]])
                g2 = plsc.load_gather(c_v, [buf[pl.ds(p + 2 * L, L)]])
                g3 = plsc.load_gather(c_v, [buf[pl.ds(p + 3 * L, L)]])
                return (a[0] + g0, a[1] + g1, a[2] + g2, a[3] + g3)

            accs = _acc
        total = accs[0] + accs[1] + accs[2] + accs[3]
        part_vec = plsc.load_gather(
            part_v, [jnp.full((L,), r0 + rloc, jnp.int32)])
        res_v[rloc, :] = jnp.full((L,), jnp.sum(total), jnp.float32) + part_vec
    pltpu.sync_copy(res_v, out_hbm.at[pl.ds(r0, ROWS_PER_W)])


def kernel(idx, noise, centers_raw, log_scales, theta_raw, logits):
    del centers_raw, theta_raw
    noise_v = (noise.reshape(B, N // 128, 128, 2)
               .transpose(0, 1, 3, 2)
               .reshape(B, 2 * N // 128, 128))
    c, part = _tc_call(log_scales[:, 0], log_scales[:, 1], logits, noise_v)
    out2d = _sc_gather(c, part, idx)
    return out2d[:, 0]


# speedup vs baseline: 848.0686x; 1.1321x over previous
import functools
import math

import jax
import jax.numpy as jnp
from jax import lax
from jax.experimental import pallas as pl
from jax.experimental.pallas import tpu as pltpu
from jax.experimental.pallas import tpu_sc as plsc

K = 64
B = 64
N = 32768
MIN_SCALE = 0.0005
LOG_2PI = math.log(2.0 * math.pi)

NC = 2
NS = 16
L = 16
NW = NC * NS
ROWS_PER_W = B // NW
CHUNK = 8192
CPR = N // CHUNK
NCH = ROWS_PER_W * CPR

CBLK = 8192
_RB = CBLK // 128


def _prep_body(ls0_ref, ls1_ref, logits_ref, c_ref, base_ref):
    ls0 = ls0_ref[...]
    ls1 = ls1_ref[...]
    lg = logits_ref[...]
    sp0 = jnp.maximum(ls0, 0.0) + jnp.log(1.0 + jnp.exp(-jnp.abs(ls0))) + MIN_SCALE
    sp1 = jnp.maximum(ls1, 0.0) + jnp.log(1.0 + jnp.exp(-jnp.abs(ls1))) + MIN_SCALE
    logdet = jnp.log(sp0 * sp1 + 1e-12)
    e = jnp.exp(lg - jnp.max(lg))
    w = e / jnp.sum(e)
    logw = jnp.log(w + 1e-9)
    c_ref[...] = logw - logdet
    base_ref[...] = jnp.full((B,), -N * LOG_2PI, jnp.float32)


_prep_call = pl.pallas_call(
    _prep_body,
    out_shape=[
        jax.ShapeDtypeStruct((K,), jnp.float32),
        jax.ShapeDtypeStruct((B,), jnp.float32),
    ],
)


def _tc_body(noise_ref, part_ref):
    i = pl.program_id(0)

    @pl.when(i == 0)
    def _init():
        part_ref[...] = jnp.zeros((B,), jnp.float32)

    x = noise_ref[...]
    part_ref[...] = part_ref[...] - 0.5 * jnp.sum(x * x, axis=(1, 2))


_tc_call = pl.pallas_call(
    _tc_body,
    grid=(2 * N // CBLK,),
    in_specs=[pl.BlockSpec((B, _RB, 128), lambda i: (0, i, 0))],
    out_specs=pl.BlockSpec((B,), lambda i: (0,)),
    out_shape=jax.ShapeDtypeStruct((B,), jnp.float32),
)

_sc_mesh = plsc.VectorSubcoreMesh(core_axis_name="c", subcore_axis_name="s")


@functools.partial(
    pl.kernel,
    out_type=jax.ShapeDtypeStruct((B, L), jnp.float32),
    mesh=_sc_mesh,
    compiler_params=pltpu.CompilerParams(needs_layout_passes=False),
    scratch_types=[
        pltpu.VMEM((CHUNK,), jnp.int32),
        pltpu.VMEM((CHUNK,), jnp.int32),
        pltpu.VMEM((K,), jnp.float32),
        pltpu.VMEM((B,), jnp.float32),
        pltpu.VMEM((ROWS_PER_W, L), jnp.float32),
        pltpu.SemaphoreType.DMA,
        pltpu.SemaphoreType.DMA,
    ],
)
def _sc_gather(c_hbm, part_hbm, idx_hbm, out_hbm,
               buf0, buf1, c_v, part_v, res_v, sem0, sem1):
    wid = lax.axis_index("s") * NC + lax.axis_index("c")
    r0 = wid * ROWS_PER_W

    pltpu.sync_copy(c_hbm, c_v)
    pltpu.sync_copy(part_hbm, part_v)

    bufs = (buf0, buf1)
    sems = (sem0, sem1)

    def start(t):
        r = r0 + (t // CPR)
        off = (t % CPR) * CHUNK
        return pltpu.async_copy(idx_hbm.at[r, pl.ds(off, CHUNK)],
                                bufs[t % 2], sems[t % 2])

    handles = {0: start(0)}
    for rloc in range(ROWS_PER_W):
        zero = jnp.zeros((L,), jnp.float32)
        accs = (zero, zero, zero, zero)
        for kk in range(CPR):
            t = rloc * CPR + kk
            if t + 1 < NCH:
                handles[t + 1] = start(t + 1)
            handles[t].wait()
            buf = bufs[t % 2]

            @plsc.parallel_loop(0, CHUNK, step=4 * L, unroll=2, carry=accs)
            def _acc(p, a):
                g0 = plsc.load_gather(c_v, [buf[pl.ds(p, L)]])
                g1 = plsc.load_gather(c_v, [buf[pl.ds(p + L, L)]])
                g2 = plsc.load_gather(c_v, [buf[pl.ds(p + 2 * L, L)]])
                g3 = plsc.load_gather(c_v, [buf[pl.ds(p + 3 * L, L)]])
                return (a[0] + g0, a[1] + g1, a[2] + g2, a[3] + g3)

            accs = _acc
        total = accs[0] + accs[1] + accs[2] + accs[3]
        part_vec = plsc.load_gather(
            part_v, [jnp.full((L,), r0 + rloc, jnp.int32)])
        res_v[rloc, :] = jnp.full((L,), jnp.sum(total), jnp.float32) + part_vec
    pltpu.sync_copy(res_v, out_hbm.at[pl.ds(r0, ROWS_PER_W)])


def kernel(idx, noise, centers_raw, log_scales, theta_raw, logits):
    del centers_raw, theta_raw
    noise_v = (noise.reshape(B, N // 128, 128, 2)
               .transpose(0, 1, 3, 2)
               .reshape(B, 2 * N // 128, 128))
    c, base = _prep_call(log_scales[:, 0], log_scales[:, 1], logits)
    npart = _tc_call(noise_v)
    out2d = _sc_gather(c, base, idx)
    return out2d[:, 0] + npart
